# baseline (speedup 1.0000x reference)
import functools
import math

import jax
import jax.numpy as jnp
import numpy as np
from jax import lax
from jax.experimental import pallas as pl
from jax.experimental.pallas import tpu as pltpu
from jax.experimental.pallas import tpu_sc as plsc

VOCAB = 1000000
EMB = 64
MAX_LEN = 512
B = 4096
L = 200

U = 100
B2 = B * L // U
NC, NS = 2, 16
NW = NC * NS
UPW = B2 // NW
NBUF = 4
LANES = 16


def _make_pe():
    pe = np.zeros((MAX_LEN, EMB), dtype=np.float32)
    position = np.arange(0, MAX_LEN, dtype=np.float32)[:, None]
    div_term = np.exp(
        np.arange(0, EMB, 2, dtype=np.float32) * -(math.log(10000.0) / EMB))
    pe[:, 0::2] = np.sin(position * div_term)
    pe[:, 1::2] = np.cos(position * div_term)
    return pe[:L]


_PE = _make_pe()


def _sc_body(inp_hbm, table_hbm, pe_hbm, out_hbm,
             idx_all, pe_v, gbufs, obufs, gsems, osems):
    wid = lax.axis_index("s") * NC + lax.axis_index("c")
    base = wid * UPW

    pltpu.sync_copy(inp_hbm.at[pl.ds(base, UPW)], idx_all)
    pltpu.sync_copy(pe_hbm, pe_v)

    for s in range(NBUF):
        pltpu.async_copy(table_hbm.at[idx_all.at[s]], gbufs[s], gsems[s])

    @pl.loop(0, UPW, step=NBUF)
    def _outer(o):
        for s in range(NBUF):
            lt = o + s
            pltpu.make_async_copy(
                table_hbm.at[idx_all.at[lt]], gbufs[s], gsems[s]).wait()

            @pl.when(o > 0)
            def _wait_prev_out():
                pltpu.make_async_copy(
                    obufs[s], out_hbm.at[base + lt - NBUF], osems[s]).wait()

            g = gbufs[s]
            ob = obufs[s]
            poff = (s & 1) * U

            @pl.loop(0, U)
            def _row(i):
                for j in range(EMB // LANES):
                    sl = pl.ds(j * LANES, LANES)
                    ob[i, sl] = g[i, sl] * 8.0 + pe_v[poff + i, sl]

            pltpu.async_copy(ob, out_hbm.at[base + lt], osems[s])

            @pl.when(lt + NBUF < UPW)
            def _next_gather():
                pltpu.async_copy(
                    table_hbm.at[idx_all.at[lt + NBUF]], gbufs[s], gsems[s])

    for s in range(NBUF):
        pltpu.make_async_copy(
            obufs[s], out_hbm.at[base + UPW - NBUF + s], osems[s]).wait()


@functools.partial(jax.jit, static_argnames=())
def _sc_call(idx2d, table, pe):
    mesh = plsc.VectorSubcoreMesh(core_axis_name="c", subcore_axis_name="s")

    def body(inp_hbm, table_hbm, pe_hbm, out_hbm,
             idx_all, pe_v,
             g0, g1, g2, g3, o0, o1, o2, o3,
             gs0, gs1, gs2, gs3, os0, os1, os2, os3):
        _sc_body(inp_hbm, table_hbm, pe_hbm, out_hbm, idx_all, pe_v,
                 [g0, g1, g2, g3], [o0, o1, o2, o3],
                 [gs0, gs1, gs2, gs3], [os0, os1, os2, os3])

    f = pl.kernel(
        body,
        out_type=jax.ShapeDtypeStruct((B2, U, EMB), jnp.float32),
        mesh=mesh,
        scratch_types=[
            pltpu.VMEM((UPW, U), jnp.int32),
            pltpu.VMEM((L, EMB), jnp.float32),
        ] + [pltpu.VMEM((U, EMB), jnp.float32) for _ in range(2 * NBUF)]
          + [pltpu.SemaphoreType.DMA for _ in range(2 * NBUF)],
        compiler_params=pltpu.CompilerParams(use_tc_tiling_on_sc=False),
    )
    return f(idx2d, table, pe)


def kernel(input, table):
    idx2d = jnp.asarray(input, jnp.int32).reshape(B2, U)
    pe = jnp.asarray(_PE)
    out = _sc_call(idx2d, table, pe)
    return out.reshape(B, L, EMB)


# speedup vs baseline: 1.0009x; 1.0009x over previous
import functools
import math

import jax
import jax.numpy as jnp
import numpy as np
from jax import lax
from jax.experimental import pallas as pl
from jax.experimental.pallas import tpu as pltpu
from jax.experimental.pallas import tpu_sc as plsc

VOCAB = 1000000
EMB = 64
MAX_LEN = 512
B = 4096
L = 200

U = 100
B2 = B * L // U
NC, NS = 2, 16
NW = NC * NS
UPW = B2 // NW
NBUF = 4
LANES = 16


def _make_pe():
    pe = np.zeros((MAX_LEN, EMB), dtype=np.float32)
    position = np.arange(0, MAX_LEN, dtype=np.float32)[:, None]
    div_term = np.exp(
        np.arange(0, EMB, 2, dtype=np.float32) * -(math.log(10000.0) / EMB))
    pe[:, 0::2] = np.sin(position * div_term)
    pe[:, 1::2] = np.cos(position * div_term)
    return pe[:L]


_PE = _make_pe()


def _sc_body(inp_hbm, table_hbm, pe_hbm, out_hbm,
             idx_all, pe_v, gbufs, obufs, gsems, osems):
    wid = lax.axis_index("s") * NC + lax.axis_index("c")
    base = wid * UPW

    pltpu.sync_copy(inp_hbm.at[pl.ds(base, UPW)], idx_all)
    pltpu.sync_copy(pe_hbm, pe_v)

    for s in range(NBUF):
        pltpu.async_copy(table_hbm.at[idx_all.at[s]], gbufs[s], gsems[s])

    base2 = wid * (UPW // 2)

    @pl.loop(0, UPW, step=NBUF)
    def _outer(o):
        for s in range(NBUF):
            lt = o + s
            b_idx = base2 + o // 2 + (s // 2)
            col = pl.ds((s & 1) * U, U)
            pltpu.make_async_copy(
                table_hbm.at[idx_all.at[lt]], gbufs[s], gsems[s]).wait()

            @pl.when(o > 0)
            def _wait_prev_out():
                pltpu.make_async_copy(
                    obufs[s], out_hbm.at[b_idx - NBUF // 2, col], osems[s]).wait()

            g = gbufs[s]
            ob = obufs[s]
            poff = (s & 1) * U

            @pl.loop(0, U)
            def _row(i):
                for j in range(EMB // LANES):
                    sl = pl.ds(j * LANES, LANES)
                    ob[i, sl] = g[i, sl] * 8.0 + pe_v[poff + i, sl]

            pltpu.async_copy(ob, out_hbm.at[b_idx, col], osems[s])

            @pl.when(lt + NBUF < UPW)
            def _next_gather():
                pltpu.async_copy(
                    table_hbm.at[idx_all.at[lt + NBUF]], gbufs[s], gsems[s])

    for s in range(NBUF):
        pltpu.make_async_copy(
            obufs[s],
            out_hbm.at[base2 + UPW // 2 - 1, pl.ds((s & 1) * U, U)],
            osems[s]).wait()


@functools.partial(jax.jit, static_argnames=())
def _sc_call(idx2d, table, pe):
    mesh = plsc.VectorSubcoreMesh(core_axis_name="c", subcore_axis_name="s")

    def body(inp_hbm, table_hbm, pe_hbm, out_hbm,
             idx_all, pe_v,
             g0, g1, g2, g3, o0, o1, o2, o3,
             gs0, gs1, gs2, gs3, os0, os1, os2, os3):
        _sc_body(inp_hbm, table_hbm, pe_hbm, out_hbm, idx_all, pe_v,
                 [g0, g1, g2, g3], [o0, o1, o2, o3],
                 [gs0, gs1, gs2, gs3], [os0, os1, os2, os3])

    f = pl.kernel(
        body,
        out_type=jax.ShapeDtypeStruct((B, L, EMB), jnp.float32),
        mesh=mesh,
        scratch_types=[
            pltpu.VMEM((UPW, U), jnp.int32),
            pltpu.VMEM((L, EMB), jnp.float32),
        ] + [pltpu.VMEM((U, EMB), jnp.float32) for _ in range(2 * NBUF)]
          + [pltpu.SemaphoreType.DMA for _ in range(2 * NBUF)],
        compiler_params=pltpu.CompilerParams(use_tc_tiling_on_sc=False),
    )
    return f(idx2d, table, pe)


def kernel(input, table):
    idx2d = jnp.asarray(input, jnp.int32).reshape(B2, U)
    pe = jnp.asarray(_PE)
    return _sc_call(idx2d, table, pe)
